# Initial kernel scaffold; baseline (speedup 1.0000x reference)
import jax
import jax.numpy as jnp
from jax.experimental import pallas as pl


def kernel(x, W1, b1, W2, b2, W3, b3, Wih, Whh, bih, bhh, Wfc, bfc, edge_index):
    raise NotImplementedError("write your pallas kernel here")


# speedup vs baseline: 15.0419x; 15.0419x over previous
import functools

import jax
import jax.numpy as jnp
from jax import lax
from jax.experimental import pallas as pl
from jax.experimental.pallas import tpu as pltpu
from jax.experimental.pallas import tpu_sc as plsc

NC = 2
NS = 16
NW = NC * NS
CB = 128
CH = 80
NR = 10240
JUNK = NR - 10000


def _sc_mesh():
    return plsc.VectorSubcoreMesh(
        core_axis_name="c", subcore_axis_name="s", num_cores=NC,
        num_subcores=NS)


def _sc_degree(dst3, ones16, z16):
    rows_per_tile = NR // NS
    zsteps = rows_per_tile // CB

    @functools.partial(
        pl.kernel,
        out_type=jax.ShapeDtypeStruct((NC, NR, 16), jnp.float32),
        mesh=_sc_mesh(),
        scratch_types=[
            pltpu.VMEM((CH, CB), jnp.int32),
            pltpu.VMEM((CB, 16), jnp.float32),
            pltpu.VMEM_SHARED((NR, 16), jnp.float32),
        ],
    )
    def k(dst_hbm, ones_hbm, z_hbm, out_hbm, dst_v, ones_v, acc):
        cid = lax.axis_index("c")
        sid = lax.axis_index("s")
        wid = cid * NS + sid
        for j in range(zsteps):
            pltpu.sync_copy(
                z_hbm, acc.at[pl.ds((sid * zsteps + j) * CB, CB), :])
        pltpu.sync_copy(dst_hbm.at[wid], dst_v)
        pltpu.sync_copy(ones_hbm, ones_v)
        plsc.subcore_barrier()

        def body(j, carry):
            pltpu.sync_copy(ones_v, acc.at[dst_v.at[j]], add=True)
            return carry

        lax.fori_loop(0, CH, body, 0)
        plsc.subcore_barrier()
        pltpu.sync_copy(
            acc.at[pl.ds(sid * rows_per_tile, rows_per_tile), :],
            out_hbm.at[cid, pl.ds(sid * rows_per_tile, rows_per_tile), :])

    return k(dst3, ones16, z16)


def _sc_scatter(hs, src3, dst3, zrow, d):
    rows_per_tile = NR // NS
    zsteps = rows_per_tile // CB

    @functools.partial(
        pl.kernel,
        out_type=jax.ShapeDtypeStruct((NC, NR, d), jnp.float32),
        mesh=_sc_mesh(),
        scratch_types=[
            pltpu.VMEM((CH, CB), jnp.int32),
            pltpu.VMEM((CH, CB), jnp.int32),
            pltpu.VMEM((CB, d), jnp.float32),
            pltpu.VMEM_SHARED((NR, d), jnp.float32),
        ],
    )
    def k(hs_hbm, src_hbm, dst_hbm, z_hbm, out_hbm,
          src_v, dst_v, buf, acc):
        cid = lax.axis_index("c")
        sid = lax.axis_index("s")
        wid = cid * NS + sid
        for j in range(zsteps):
            pltpu.sync_copy(
                z_hbm, acc.at[pl.ds((sid * zsteps + j) * CB, CB), :])
        pltpu.sync_copy(src_hbm.at[wid], src_v)
        pltpu.sync_copy(dst_hbm.at[wid], dst_v)
        plsc.subcore_barrier()

        def body(j, carry):
            pltpu.sync_copy(hs_hbm.at[src_v.at[j]], buf)
            pltpu.sync_copy(buf, acc.at[dst_v.at[j]], add=True)
            return carry

        lax.fori_loop(0, CH, body, 0)
        plsc.subcore_barrier()
        pltpu.sync_copy(
            acc.at[pl.ds(sid * rows_per_tile, rows_per_tile), :],
            out_hbm.at[cid, pl.ds(sid * rows_per_tile, rows_per_tile), :])

    return k(hs, src3, dst3, zrow)


def _tc_first(x2, W1, degp):
    n, din = x2.shape
    dout = W1.shape[1]
    rb = 2000
    grid = n // rb

    def body(x_ref, w_ref, deg_ref, o_ref):
        dg = deg_ref[...]
        dinv = lax.rsqrt(dg[0] + dg[1] + 1.0)[:, 0:1]
        o_ref[...] = jnp.dot(x_ref[...], w_ref[...],
                             preferred_element_type=jnp.float32) * dinv

    return pl.pallas_call(
        body,
        grid=(grid,),
        in_specs=[
            pl.BlockSpec((rb, din), lambda i: (i, 0)),
            pl.BlockSpec((din, dout), lambda i: (0, 0)),
            pl.BlockSpec((NC, rb, 16), lambda i: (0, i, 0)),
        ],
        out_specs=pl.BlockSpec((rb, dout), lambda i: (i, 0)),
        out_shape=jax.ShapeDtypeStruct((n, dout), jnp.float32),
    )(x2, W1, degp)


def _tc_combine(part, hsl, b_row, W, degp, relu, post_scale, post_bias=None):
    n, d = hsl.shape
    dout = W.shape[1]
    rb = 2000
    grid = n // rb
    has_pb = post_bias is not None

    def body(p_ref, h_ref, b_ref, w_ref, deg_ref, *rest):
        if has_pb:
            pb_ref, o_ref = rest
        else:
            (o_ref,) = rest
        dg = deg_ref[...]
        dinv = lax.rsqrt(dg[0] + dg[1] + 1.0)[:, 0:1]
        p = p_ref[...]
        comb = (p[0] + p[1] + h_ref[...]) * dinv + b_ref[...]
        if relu:
            comb = jnp.maximum(comb, 0.0)
        out = jnp.dot(comb, w_ref[...], preferred_element_type=jnp.float32)
        if post_scale:
            out = out * dinv
        if has_pb:
            out = out + pb_ref[...]
        o_ref[...] = out

    in_specs = [
        pl.BlockSpec((NC, rb, d), lambda i: (0, i, 0)),
        pl.BlockSpec((rb, d), lambda i: (i, 0)),
        pl.BlockSpec((1, d), lambda i: (0, 0)),
        pl.BlockSpec((d, dout), lambda i: (0, 0)),
        pl.BlockSpec((NC, rb, 16), lambda i: (0, i, 0)),
    ]
    args = [part, hsl, b_row, W, degp]
    if has_pb:
        in_specs.append(pl.BlockSpec((1, dout), lambda i: (0, 0)))
        args.append(post_bias)

    return pl.pallas_call(
        body,
        grid=(grid,),
        in_specs=in_specs,
        out_specs=pl.BlockSpec((rb, dout), lambda i: (i, 0)),
        out_shape=jax.ShapeDtypeStruct((n, dout), jnp.float32),
    )(*args)


def _tc_lstm(xproj, Whh_t, Wfc_t, bfc_row):
    t_steps, b, g = xproj.shape
    h = g // 4

    def body(xp_ref, whh_ref, wfc_ref, bfc_ref, o_ref, hh, cc):
        t = pl.program_id(0)

        @pl.when(t == 0)
        def _():
            hh[...] = jnp.zeros_like(hh)
            cc[...] = jnp.zeros_like(cc)

        gates = xp_ref[0] + jnp.dot(
            hh[...], whh_ref[...], preferred_element_type=jnp.float32)
        gi = jax.nn.sigmoid(gates[:, 0:h])
        gf = jax.nn.sigmoid(gates[:, h:2 * h])
        gg = jnp.tanh(gates[:, 2 * h:3 * h])
        go = jax.nn.sigmoid(gates[:, 3 * h:4 * h])
        c2 = gf * cc[...] + gi * gg
        h2 = go * jnp.tanh(c2)
        cc[...] = c2
        hh[...] = h2

        @pl.when(t == t_steps - 1)
        def _():
            o_ref[...] = jnp.dot(
                h2, wfc_ref[...],
                preferred_element_type=jnp.float32) + bfc_ref[...]

    return pl.pallas_call(
        body,
        grid=(t_steps,),
        in_specs=[
            pl.BlockSpec((1, b, g), lambda t: (t, 0, 0)),
            pl.BlockSpec((h, g), lambda t: (0, 0)),
            pl.BlockSpec((h, h), lambda t: (0, 0)),
            pl.BlockSpec((1, h), lambda t: (0, 0)),
        ],
        out_specs=pl.BlockSpec((b, h), lambda t: (0, 0)),
        out_shape=jax.ShapeDtypeStruct((b, h), jnp.float32),
        scratch_shapes=[
            pltpu.VMEM((b, h), jnp.float32),
            pltpu.VMEM((b, h), jnp.float32),
        ],
    )(xproj, Whh_t, Wfc_t, bfc_row)


def kernel(x, W1, b1, W2, b2, W3, b3, Wih, Whh, bih, bhh, Wfc, bfc,
           edge_index):
    B, T, IN = x.shape
    n = B * T
    x2 = x.reshape(n, IN)

    ei = edge_index.astype(jnp.int32)
    src, dst = ei[0], ei[1]
    e = src.shape[0]
    ep = NW * CH * CB
    pad = ep - e
    apad = jnp.arange(pad, dtype=jnp.int32)
    src3 = jnp.concatenate([src, (apad * 37) % n]).reshape(NW, CH, CB)
    dst3 = jnp.concatenate([dst, n + apad % JUNK]).reshape(NW, CH, CB)

    ones16 = jnp.ones((CB, 16), jnp.float32)
    z16 = jnp.zeros((CB, 16), jnp.float32)
    z128 = jnp.zeros((CB, 128), jnp.float32)

    W1p = jnp.concatenate([W1, jnp.zeros((IN, 64), jnp.float32)], axis=1)
    b1p = jnp.concatenate([b1, jnp.zeros((64,), jnp.float32)])
    W2p = jnp.concatenate([W2, jnp.zeros((64, 128), jnp.float32)], axis=0)

    degp = _sc_degree(dst3, ones16, z16)
    hs1 = _tc_first(x2, W1p, degp)
    p1 = _sc_scatter(hs1, src3, dst3, z128, 128)
    hs2 = _tc_combine(p1, hs1, b1p.reshape(1, -1), W2p, degp,
                      relu=True, post_scale=True)
    p2 = _sc_scatter(hs2, src3, dst3, z128, 128)
    hs3 = _tc_combine(p2, hs2, b2.reshape(1, -1), W3, degp,
                      relu=True, post_scale=True)
    p3 = _sc_scatter(hs3, src3, dst3, z128, 128)
    xproj = _tc_combine(p3, hs3, b3.reshape(1, -1), Wih.T, degp,
                        relu=False, post_scale=False,
                        post_bias=(bih + bhh).reshape(1, -1))

    return _tc_lstm(jnp.swapaxes(xproj.reshape(B, T, 512), 0, 1), Whh.T,
                    Wfc.T, bfc.reshape(1, -1))


# speedup vs baseline: 18.0015x; 1.1968x over previous
import functools

import jax
import jax.numpy as jnp
from jax import lax
from jax.experimental import pallas as pl
from jax.experimental.pallas import tpu as pltpu
from jax.experimental.pallas import tpu_sc as plsc

NC = 2
NS = 16
NW = NC * NS
CB = 128
CH = 80
KSUP = 8
NSUP = CH // KSUP
NR = 10240
JUNK = NR - 10000


def _sc_mesh():
    return plsc.VectorSubcoreMesh(
        core_axis_name="c", subcore_axis_name="s", num_cores=NC,
        num_subcores=NS)


def _sc_degree(dst3, ones16, z16):
    rows_per_tile = NR // NS
    zsteps = rows_per_tile // CB

    @functools.partial(
        pl.kernel,
        out_type=jax.ShapeDtypeStruct((NC, NR, 16), jnp.float32),
        mesh=_sc_mesh(),
        scratch_types=[
            pltpu.VMEM((CH, CB), jnp.int32),
            pltpu.VMEM((CB, 16), jnp.float32),
            pltpu.VMEM_SHARED((NR, 16), jnp.float32),
        ],
    )
    def k(dst_hbm, ones_hbm, z_hbm, out_hbm, dst_v, ones_v, acc):
        cid = lax.axis_index("c")
        sid = lax.axis_index("s")
        wid = cid * NS + sid
        for j in range(zsteps):
            pltpu.sync_copy(
                z_hbm, acc.at[pl.ds((sid * zsteps + j) * CB, CB), :])
        pltpu.sync_copy(dst_hbm.at[wid], dst_v)
        pltpu.sync_copy(ones_hbm, ones_v)
        plsc.subcore_barrier()

        def body(j, carry):
            pltpu.sync_copy(ones_v, acc.at[dst_v.at[j]], add=True)
            return carry

        lax.fori_loop(0, CH, body, 0)
        plsc.subcore_barrier()
        pltpu.sync_copy(
            acc.at[pl.ds(sid * rows_per_tile, rows_per_tile), :],
            out_hbm.at[cid, pl.ds(sid * rows_per_tile, rows_per_tile), :])

    return k(dst3, ones16, z16)


def _sc_scatter(hs, src4, dst3, zrow, d):
    rows_per_tile = NR // NS
    zsteps = rows_per_tile // CB
    pairs = NSUP // 2

    @functools.partial(
        pl.kernel,
        out_type=jax.ShapeDtypeStruct((NC, NR, d), jnp.float32),
        mesh=_sc_mesh(),
        scratch_types=[
            pltpu.VMEM((CH, CB), jnp.int32),
            pltpu.VMEM((2 * KSUP, CB), jnp.int32),
            pltpu.VMEM((2, CB, d), jnp.float32),
            pltpu.VMEM_SHARED((NR, d), jnp.float32),
            pltpu.SemaphoreType.DMA,
            pltpu.SemaphoreType.DMA,
            pltpu.SemaphoreType.DMA,
            pltpu.SemaphoreType.DMA,
            pltpu.SemaphoreType.DMA,
            pltpu.SemaphoreType.DMA,
        ],
    )
    def k(hs_hbm, src_hbm, dst_hbm, z_hbm, out_hbm,
          dst_v, si, buf, acc, sem_i0, sem_i1, sem_d0, sem_d1,
          sem_s0, sem_s1):
        cid = lax.axis_index("c")
        sid = lax.axis_index("s")
        wid = cid * NS + sid
        for j in range(zsteps):
            pltpu.sync_copy(
                z_hbm, acc.at[pl.ds((sid * zsteps + j) * CB, CB), :])
        pltpu.sync_copy(dst_hbm.at[wid], dst_v)
        slot0 = si.at[pl.ds(0, KSUP)]
        slot1 = si.at[pl.ds(KSUP, KSUP)]
        pltpu.async_copy(src_hbm.at[wid * NSUP], slot0, sem_i0)
        pltpu.async_copy(src_hbm.at[wid * NSUP + 1], slot1, sem_i1)
        plsc.subcore_barrier()
        pltpu.make_async_copy(src_hbm.at[wid * NSUP], slot0, sem_i0).wait()
        pltpu.sync_copy(z_hbm, buf.at[1])
        pltpu.async_copy(buf.at[1], acc.at[dst_v.at[0]], sem_s1, add=True)
        pltpu.async_copy(hs_hbm.at[si.at[0]], buf.at[0], sem_d0)

        sems_d = (sem_d0, sem_d1)
        sems_s = (sem_s0, sem_s1)

        def body(s2, carry):
            base = 2 * KSUP * s2
            s_a = 2 * s2
            for t in range(2 * KSUP):
                j = base + t
                par = t % 2
                pltpu.make_async_copy(
                    hs_hbm.at[si.at[t]], buf.at[par], sems_d[par]).wait()
                pltpu.make_async_copy(
                    buf.at[1 - par], acc.at[dst_v.at[j]],
                    sems_s[1 - par]).wait()
                if t < 2 * KSUP - 1:
                    if t == KSUP - 1:
                        pltpu.make_async_copy(
                            src_hbm.at[wid * NSUP], slot1, sem_i1).wait()
                    pltpu.async_copy(hs_hbm.at[si.at[t + 1]],
                                     buf.at[1 - par], sems_d[1 - par])
                    if t == KSUP - 1:
                        sup = jnp.minimum(s_a + 2, NSUP - 1)
                        pltpu.async_copy(src_hbm.at[wid * NSUP + sup],
                                         slot0, sem_i0)
                else:
                    pltpu.make_async_copy(
                        src_hbm.at[wid * NSUP], slot0, sem_i0).wait()
                    pltpu.async_copy(hs_hbm.at[si.at[0]], buf.at[0],
                                     sem_d0)
                    sup = jnp.minimum(s_a + 3, NSUP - 1)
                    pltpu.async_copy(src_hbm.at[wid * NSUP + sup],
                                     slot1, sem_i1)
                pltpu.async_copy(buf.at[par], acc.at[dst_v.at[j]],
                                 sems_s[par], add=True)
            return carry

        lax.fori_loop(0, pairs, body, 0)
        pltpu.make_async_copy(hs_hbm.at[si.at[0]], buf.at[0],
                              sem_d0).wait()
        pltpu.make_async_copy(src_hbm.at[wid * NSUP], slot1, sem_i1).wait()
        pltpu.make_async_copy(buf.at[1], acc.at[dst_v.at[0]],
                              sem_s1).wait()
        plsc.subcore_barrier()
        pltpu.sync_copy(
            acc.at[pl.ds(sid * rows_per_tile, rows_per_tile), :],
            out_hbm.at[cid, pl.ds(sid * rows_per_tile, rows_per_tile), :])

    return k(hs, src4, dst3, zrow)


def _tc_first(x2, W1, degp):
    n, din = x2.shape
    dout = W1.shape[1]
    rb = 2000
    grid = n // rb

    def body(x_ref, w_ref, deg_ref, o_ref):
        dg = deg_ref[...]
        dinv = lax.rsqrt(dg[0] + dg[1] + 1.0)[:, 0:1]
        o_ref[...] = jnp.dot(x_ref[...], w_ref[...],
                             preferred_element_type=jnp.float32) * dinv

    return pl.pallas_call(
        body,
        grid=(grid,),
        in_specs=[
            pl.BlockSpec((rb, din), lambda i: (i, 0)),
            pl.BlockSpec((din, dout), lambda i: (0, 0)),
            pl.BlockSpec((NC, rb, 16), lambda i: (0, i, 0)),
        ],
        out_specs=pl.BlockSpec((rb, dout), lambda i: (i, 0)),
        out_shape=jax.ShapeDtypeStruct((n, dout), jnp.float32),
    )(x2, W1, degp)


def _tc_combine(part, hsl, b_row, W, degp, relu, post_scale, post_bias=None):
    n, d = hsl.shape
    dout = W.shape[1]
    rb = 2000
    grid = n // rb
    has_pb = post_bias is not None

    def body(p_ref, h_ref, b_ref, w_ref, deg_ref, *rest):
        if has_pb:
            pb_ref, o_ref = rest
        else:
            (o_ref,) = rest
        dg = deg_ref[...]
        dinv = lax.rsqrt(dg[0] + dg[1] + 1.0)[:, 0:1]
        p = p_ref[...]
        comb = (p[0] + p[1] + h_ref[...]) * dinv + b_ref[...]
        if relu:
            comb = jnp.maximum(comb, 0.0)
        out = jnp.dot(comb, w_ref[...], preferred_element_type=jnp.float32)
        if post_scale:
            out = out * dinv
        if has_pb:
            out = out + pb_ref[...]
        o_ref[...] = out

    in_specs = [
        pl.BlockSpec((NC, rb, d), lambda i: (0, i, 0)),
        pl.BlockSpec((rb, d), lambda i: (i, 0)),
        pl.BlockSpec((1, d), lambda i: (0, 0)),
        pl.BlockSpec((d, dout), lambda i: (0, 0)),
        pl.BlockSpec((NC, rb, 16), lambda i: (0, i, 0)),
    ]
    args = [part, hsl, b_row, W, degp]
    if has_pb:
        in_specs.append(pl.BlockSpec((1, dout), lambda i: (0, 0)))
        args.append(post_bias)

    return pl.pallas_call(
        body,
        grid=(grid,),
        in_specs=in_specs,
        out_specs=pl.BlockSpec((rb, dout), lambda i: (i, 0)),
        out_shape=jax.ShapeDtypeStruct((n, dout), jnp.float32),
    )(*args)


def _tc_lstm(xproj, Whh_t, Wfc_t, bfc_row):
    t_steps, b, g = xproj.shape
    h = g // 4

    def body(xp_ref, whh_ref, wfc_ref, bfc_ref, o_ref, hh, cc):
        t = pl.program_id(0)

        @pl.when(t == 0)
        def _():
            hh[...] = jnp.zeros_like(hh)
            cc[...] = jnp.zeros_like(cc)

        gates = xp_ref[0] + jnp.dot(
            hh[...], whh_ref[...], preferred_element_type=jnp.float32)
        gi = jax.nn.sigmoid(gates[:, 0:h])
        gf = jax.nn.sigmoid(gates[:, h:2 * h])
        gg = jnp.tanh(gates[:, 2 * h:3 * h])
        go = jax.nn.sigmoid(gates[:, 3 * h:4 * h])
        c2 = gf * cc[...] + gi * gg
        h2 = go * jnp.tanh(c2)
        cc[...] = c2
        hh[...] = h2

        @pl.when(t == t_steps - 1)
        def _():
            o_ref[...] = jnp.dot(
                h2, wfc_ref[...],
                preferred_element_type=jnp.float32) + bfc_ref[...]

    return pl.pallas_call(
        body,
        grid=(t_steps,),
        in_specs=[
            pl.BlockSpec((1, b, g), lambda t: (t, 0, 0)),
            pl.BlockSpec((h, g), lambda t: (0, 0)),
            pl.BlockSpec((h, h), lambda t: (0, 0)),
            pl.BlockSpec((1, h), lambda t: (0, 0)),
        ],
        out_specs=pl.BlockSpec((b, h), lambda t: (0, 0)),
        out_shape=jax.ShapeDtypeStruct((b, h), jnp.float32),
        scratch_shapes=[
            pltpu.VMEM((b, h), jnp.float32),
            pltpu.VMEM((b, h), jnp.float32),
        ],
    )(xproj, Whh_t, Wfc_t, bfc_row)


def kernel(x, W1, b1, W2, b2, W3, b3, Wih, Whh, bih, bhh, Wfc, bfc,
           edge_index):
    B, T, IN = x.shape
    n = B * T
    x2 = x.reshape(n, IN)

    ei = edge_index.astype(jnp.int32)
    src, dst = ei[0], ei[1]
    e = src.shape[0]
    ep = NW * CH * CB
    pad = ep - e
    apad = jnp.arange(pad, dtype=jnp.int32)
    src4 = jnp.concatenate([src, (apad * 37) % n]).reshape(
        NW * NSUP, KSUP, CB)
    dst3 = jnp.concatenate([dst, n + apad % JUNK]).reshape(NW, CH, CB)

    ones16 = jnp.ones((CB, 16), jnp.float32)
    z16 = jnp.zeros((CB, 16), jnp.float32)
    z128 = jnp.zeros((CB, 128), jnp.float32)

    W1p = jnp.concatenate([W1, jnp.zeros((IN, 64), jnp.float32)], axis=1)
    b1p = jnp.concatenate([b1, jnp.zeros((64,), jnp.float32)])
    W2p = jnp.concatenate([W2, jnp.zeros((64, 128), jnp.float32)], axis=0)

    degp = _sc_degree(dst3, ones16, z16)
    hs1 = _tc_first(x2, W1p, degp)
    p1 = _sc_scatter(hs1, src4, dst3, z128, 128)
    hs2 = _tc_combine(p1, hs1, b1p.reshape(1, -1), W2p, degp,
                      relu=True, post_scale=True)
    p2 = _sc_scatter(hs2, src4, dst3, z128, 128)
    hs3 = _tc_combine(p2, hs2, b2.reshape(1, -1), W3, degp,
                      relu=True, post_scale=True)
    p3 = _sc_scatter(hs3, src4, dst3, z128, 128)
    xproj = _tc_combine(p3, hs3, b3.reshape(1, -1), Wih.T, degp,
                        relu=False, post_scale=False,
                        post_bias=(bih + bhh).reshape(1, -1))

    return _tc_lstm(jnp.swapaxes(xproj.reshape(B, T, 512), 0, 1), Whh.T,
                    Wfc.T, bfc.reshape(1, -1))
